# Initial kernel scaffold; baseline (speedup 1.0000x reference)
import jax
import jax.numpy as jnp
from jax.experimental import pallas as pl


def kernel(desc0, desc1, W0, q0, k0, bconv0, linW0, linb0, gamma0, beta0, W1, q1, k1, bconv1, linW1, linb1, gamma1, beta1):
    raise NotImplementedError("write your pallas kernel here")


# speedup vs baseline: 2788.2322x; 2788.2322x over previous
import functools

import jax
import jax.numpy as jnp
from jax import lax
from jax.experimental import pallas as pl

B = 4
S = 192
N = 2 * S
F = 128
TOT = B * N
NEG_SLOPE = 0.2
EPS = 1e-5


def _layer(x, w0, w1, lin_a, lin_b, vecs, same, notdiag):
    qv = vecs[0:1, :]
    kv = vecs[1:2, :]
    bconv = vecs[2:3, :]
    linb = vecs[3:4, :]
    gamma = vecs[4:5, :]
    beta = vecs[5:6, :]

    xw0 = jnp.dot(x, w0, preferred_element_type=jnp.float32)
    xw1 = jnp.dot(x, w1, preferred_element_type=jnp.float32)

    dn = (((1,), (1,)), ((), ()))
    aggr_parts = []
    for b in range(B):
        xw0b = lax.slice(xw0, (b * N, 0), ((b + 1) * N, F))
        xw1b = lax.slice(xw1, (b * N, 0), ((b + 1) * N, F))
        qi0 = lax.dot_general(xw0b, qv, dn,
                              preferred_element_type=jnp.float32)
        qi1 = lax.dot_general(xw1b, qv, dn,
                              preferred_element_type=jnp.float32)
        kj0 = lax.dot_general(kv, xw0b, dn,
                              preferred_element_type=jnp.float32)
        kj1 = lax.dot_general(kv, xw1b, dn,
                              preferred_element_type=jnp.float32)
        logits = jnp.where(same, qi0 + kj0, qi1 + kj1)
        logits = jnp.where(logits >= 0.0, logits, NEG_SLOPE * logits)
        amax = jnp.max(jnp.where(notdiag, logits, -1e30), axis=1,
                       keepdims=True)
        e = jnp.where(notdiag, jnp.exp(logits - amax), 0.0)
        denom = jnp.sum(e, axis=1, keepdims=True)
        p = e / (denom + 1e-16)
        p0 = jnp.where(same, p, 0.0)
        p1 = jnp.where(same, 0.0, p)
        aggr_parts.append(
            jnp.dot(p0, xw0b, preferred_element_type=jnp.float32)
            + jnp.dot(p1, xw1b, preferred_element_type=jnp.float32))
    aggr = jnp.concatenate(aggr_parts, axis=0)

    msg1 = jnp.maximum(aggr + bconv, 0.0)
    msg2 = (jnp.dot(x, lin_a, preferred_element_type=jnp.float32)
            + jnp.dot(msg1, lin_b, preferred_element_type=jnp.float32)
            + linb)
    mean = jnp.sum(msg2, axis=0, keepdims=True) * (1.0 / TOT)
    xc = msg2 - mean
    var = jnp.sum(xc * xc, axis=0, keepdims=True) * (1.0 / TOT)
    msg3 = xc * lax.rsqrt(var + EPS) * gamma + beta
    return x + msg3


def _rgat_kernel(x_ref,
                 w0_0_ref, w1_0_ref, lina_0_ref, linb_0_ref, vecs_0_ref,
                 w0_1_ref, w1_1_ref, lina_1_ref, linb_1_ref, vecs_1_ref,
                 out_ref):
    row = lax.broadcasted_iota(jnp.int32, (N, N), 0)
    col = lax.broadcasted_iota(jnp.int32, (N, N), 1)
    same = (row < S) == (col < S)
    notdiag = row != col

    x = x_ref[...]
    x = _layer(x, w0_0_ref[...], w1_0_ref[...], lina_0_ref[...],
               linb_0_ref[...], vecs_0_ref[...], same, notdiag)
    x = _layer(x, w0_1_ref[...], w1_1_ref[...], lina_1_ref[...],
               linb_1_ref[...], vecs_1_ref[...], same, notdiag)
    out_ref[...] = x


@functools.partial(jax.jit, static_argnames=())
def kernel(desc0, desc1, W0, q0, k0, bconv0, linW0, linb0, gamma0, beta0,
           W1, q1, k1, bconv1, linW1, linb1, gamma1, beta1):
    x = jnp.concatenate([desc0, desc1], axis=2)
    x = jnp.transpose(x, (0, 2, 1)).reshape(TOT, F)

    def pack(q, k, bconv, linb, gamma, beta):
        v = jnp.stack([q[:, 0], k[:, 0], bconv, linb, gamma, beta], axis=0)
        return jnp.pad(v, ((0, 2), (0, 0)))

    vecs0 = pack(q0, k0, bconv0, linb0, gamma0, beta0)
    vecs1 = pack(q1, k1, bconv1, linb1, gamma1, beta1)
    linT0 = linW0.T
    linT1 = linW1.T

    out = pl.pallas_call(
        _rgat_kernel,
        out_shape=jax.ShapeDtypeStruct((TOT, F), jnp.float32),
    )(x,
      W0[0], W0[1], linT0[:F], linT0[F:], vecs0,
      W1[0], W1[1], linT1[:F], linT1[F:], vecs1)

    out = out.reshape(B, N, F).transpose(0, 2, 1)
    return out[:, :, :S], out[:, :, S:]
